# baseline (speedup 1.0000x reference)
import functools

import jax
import jax.numpy as jnp
from jax import lax
from jax.experimental import pallas as pl
from jax.experimental.pallas import tpu as pltpu
from jax.experimental.pallas import tpu_sc as plsc

BATCH = 16384
EMB = 64
HID = 128
NC = 2
NS = 16
NW = NC * NS
B_PER_W = BATCH // NW
CHUNK = 256


def _sc_gather4(user, item, gmf_user, gmf_item, mlp_user, mlp_item):
    mesh = plsc.VectorSubcoreMesh(core_axis_name="c", subcore_axis_name="s")
    out_sd = jax.ShapeDtypeStruct((BATCH, EMB), jnp.float32)

    @functools.partial(
        pl.kernel,
        mesh=mesh,
        out_type=[out_sd, out_sd, out_sd, out_sd],
        compiler_params=pltpu.CompilerParams(use_tc_tiling_on_sc=False),
        scratch_types=[
            pltpu.VMEM((B_PER_W,), jnp.int32),
            pltpu.VMEM((B_PER_W,), jnp.int32),
            pltpu.VMEM((CHUNK, EMB), jnp.float32),
            pltpu.VMEM((CHUNK, EMB), jnp.float32),
            pltpu.VMEM((CHUNK, EMB), jnp.float32),
            pltpu.VMEM((CHUNK, EMB), jnp.float32),
            pltpu.SemaphoreType.DMA,
        ],
    )
    def sc_kernel(u_hbm, i_hbm, gu_hbm, gi_hbm, mu_hbm, mi_hbm,
                  ogu_hbm, ogi_hbm, omu_hbm, omi_hbm,
                  uidx_v, iidx_v, bgu, bgi, bmu, bmi, sem):
        wid = lax.axis_index("s") * NC + lax.axis_index("c")
        base = wid * B_PER_W
        pltpu.sync_copy(u_hbm.at[pl.ds(base, B_PER_W)], uidx_v)
        pltpu.sync_copy(i_hbm.at[pl.ds(base, B_PER_W)], iidx_v)
        for c in range(B_PER_W // CHUNK):
            off = c * CHUNK
            u_sl = uidx_v.at[pl.ds(off, CHUNK)]
            i_sl = iidx_v.at[pl.ds(off, CHUNK)]
            cps = [
                pltpu.async_copy(gu_hbm.at[u_sl], bgu, sem),
                pltpu.async_copy(gi_hbm.at[i_sl], bgi, sem),
                pltpu.async_copy(mu_hbm.at[u_sl], bmu, sem),
                pltpu.async_copy(mi_hbm.at[i_sl], bmi, sem),
            ]
            for cp in cps:
                cp.wait()
            dst = pl.ds(base + off, CHUNK)
            pltpu.sync_copy(bgu, ogu_hbm.at[dst])
            pltpu.sync_copy(bgi, ogi_hbm.at[dst])
            pltpu.sync_copy(bmu, omu_hbm.at[dst])
            pltpu.sync_copy(bmi, omi_hbm.at[dst])

    return sc_kernel(user, item, gmf_user, gmf_item, mlp_user, mlp_item)


def _tc_body(gu, gi, mu, mi, w1, b1, w2, b2, w3, b3, wo, bo, out):
    f32 = jnp.float32
    gmf = gu[...] * gi[...]
    h = (jnp.dot(mu[...], w1[0:EMB, :], preferred_element_type=f32)
         + jnp.dot(mi[...], w1[EMB:2 * EMB, :], preferred_element_type=f32)
         + b1[...])
    h = jnp.maximum(h, 0.0)
    h = jnp.maximum(jnp.dot(h, w2[...], preferred_element_type=f32) + b2[...], 0.0)
    h = jnp.maximum(jnp.dot(h, w3[...], preferred_element_type=f32) + b3[...], 0.0)
    out[...] = (jnp.sum(gmf * wo[0:1, 0:EMB], axis=1)
                + jnp.sum(h * wo[0:1, EMB:2 * EMB], axis=1)
                + bo[0, 0])


def _tc_dense(gu, gi, mu, mi, W1, b1, W2, b2, W3, b3, Wo, bo):
    blk = 1024
    grid = (BATCH // blk,)
    row = lambda i: (i, 0)
    rep = lambda i: (0, 0)
    return pl.pallas_call(
        _tc_body,
        grid=grid,
        in_specs=[
            pl.BlockSpec((blk, EMB), row),
            pl.BlockSpec((blk, EMB), row),
            pl.BlockSpec((blk, EMB), row),
            pl.BlockSpec((blk, EMB), row),
            pl.BlockSpec((2 * EMB, HID), rep),
            pl.BlockSpec((1, HID), rep),
            pl.BlockSpec((HID, HID // 2), rep),
            pl.BlockSpec((1, HID // 2), rep),
            pl.BlockSpec((HID // 2, EMB), rep),
            pl.BlockSpec((1, EMB), rep),
            pl.BlockSpec((1, 2 * EMB), rep),
            pl.BlockSpec((1, 1), rep),
        ],
        out_specs=pl.BlockSpec((blk,), lambda i: (i,)),
        out_shape=jax.ShapeDtypeStruct((BATCH,), jnp.float32),
    )(gu, gi, mu, mi, W1, b1, W2, b2, W3, b3, Wo, bo)


def kernel(user, item, gmf_user, gmf_item, mlp_user, mlp_item,
           W1, b1, W2, b2, W3, b3, Wo, bo):
    user = user.astype(jnp.int32)
    item = item.astype(jnp.int32)
    gu, gi, mu, mi = _sc_gather4(user, item, gmf_user, gmf_item,
                                 mlp_user, mlp_item)
    out = _tc_dense(gu, gi, mu, mi,
                    W1, b1.reshape(1, HID),
                    W2, b2.reshape(1, HID // 2),
                    W3, b3.reshape(1, EMB),
                    Wo.reshape(1, 2 * EMB), bo.reshape(1, 1))
    return out


# speedup vs baseline: 1.5126x; 1.5126x over previous
import functools

import jax
import jax.numpy as jnp
from jax import lax
from jax.experimental import pallas as pl
from jax.experimental.pallas import tpu as pltpu
from jax.experimental.pallas import tpu_sc as plsc

BATCH = 16384
EMB = 64
HID = 128
NC = 2
NS = 16
NW = NC * NS
B_PER_W = BATCH // NW
CHUNK = 128


def _sc_gather4(user, item, gmf_user, gmf_item, mlp_user, mlp_item):
    mesh = plsc.VectorSubcoreMesh(core_axis_name="c", subcore_axis_name="s")
    out_sd = jax.ShapeDtypeStruct((BATCH, EMB), jnp.float32)

    @functools.partial(
        pl.kernel,
        mesh=mesh,
        out_type=[out_sd, out_sd, out_sd, out_sd],
        scratch_types=[
            pltpu.VMEM((B_PER_W,), jnp.int32),
            pltpu.VMEM((B_PER_W,), jnp.int32),
            pltpu.VMEM((CHUNK, EMB), jnp.float32),
            pltpu.VMEM((CHUNK, EMB), jnp.float32),
            pltpu.VMEM((CHUNK, EMB), jnp.float32),
            pltpu.VMEM((CHUNK, EMB), jnp.float32),
            pltpu.SemaphoreType.DMA,
        ],
    )
    def sc_kernel(u_hbm, i_hbm, gu_hbm, gi_hbm, mu_hbm, mi_hbm,
                  ogu_hbm, ogi_hbm, omu_hbm, omi_hbm,
                  uidx_v, iidx_v, bgu, bgi, bmu, bmi, sem):
        wid = lax.axis_index("s") * NC + lax.axis_index("c")
        base = wid * B_PER_W
        pltpu.sync_copy(u_hbm.at[pl.ds(base, B_PER_W)], uidx_v)
        pltpu.sync_copy(i_hbm.at[pl.ds(base, B_PER_W)], iidx_v)
        for c in range(B_PER_W // CHUNK):
            off = c * CHUNK

            @pl.loop(0, CHUNK // 16)
            def _(g):
                uvec = uidx_v[pl.ds(off + g * 16, 16)]
                vvec = iidx_v[pl.ds(off + g * 16, 16)]
                for k in range(16):
                    u = uvec[k]
                    v = vvec[k]
                    dst = pl.ds(g * 16 + k, 1)
                    pltpu.async_copy(gu_hbm.at[pl.ds(u, 1)], bgu.at[dst], sem)
                    pltpu.async_copy(gi_hbm.at[pl.ds(v, 1)], bgi.at[dst], sem)
                    pltpu.async_copy(mu_hbm.at[pl.ds(u, 1)], bmu.at[dst], sem)
                    pltpu.async_copy(mi_hbm.at[pl.ds(v, 1)], bmi.at[dst], sem)

            pltpu.make_async_copy(gu_hbm.at[pl.ds(0, CHUNK)], bgu, sem).wait()
            pltpu.make_async_copy(gi_hbm.at[pl.ds(0, CHUNK)], bgi, sem).wait()
            pltpu.make_async_copy(mu_hbm.at[pl.ds(0, CHUNK)], bmu, sem).wait()
            pltpu.make_async_copy(mi_hbm.at[pl.ds(0, CHUNK)], bmi, sem).wait()
            dst = pl.ds(base + off, CHUNK)
            pltpu.sync_copy(bgu, ogu_hbm.at[dst])
            pltpu.sync_copy(bgi, ogi_hbm.at[dst])
            pltpu.sync_copy(bmu, omu_hbm.at[dst])
            pltpu.sync_copy(bmi, omi_hbm.at[dst])

    return sc_kernel(user, item, gmf_user, gmf_item, mlp_user, mlp_item)


def _tc_body(gu, gi, mu, mi, w1, b1, w2, b2, w3, b3, wo, bo, out):
    f32 = jnp.float32
    gmf = gu[...] * gi[...]
    h = (jnp.dot(mu[...], w1[0:EMB, :], preferred_element_type=f32)
         + jnp.dot(mi[...], w1[EMB:2 * EMB, :], preferred_element_type=f32)
         + b1[...])
    h = jnp.maximum(h, 0.0)
    h = jnp.maximum(jnp.dot(h, w2[...], preferred_element_type=f32) + b2[...], 0.0)
    h = jnp.maximum(jnp.dot(h, w3[...], preferred_element_type=f32) + b3[...], 0.0)
    out[...] = (jnp.sum(gmf * wo[0:1, 0:EMB], axis=1)
                + jnp.sum(h * wo[0:1, EMB:2 * EMB], axis=1)
                + bo[0, 0])


def _tc_dense(gu, gi, mu, mi, W1, b1, W2, b2, W3, b3, Wo, bo):
    blk = 1024
    grid = (BATCH // blk,)
    row = lambda i: (i, 0)
    rep = lambda i: (0, 0)
    return pl.pallas_call(
        _tc_body,
        grid=grid,
        in_specs=[
            pl.BlockSpec((blk, EMB), row),
            pl.BlockSpec((blk, EMB), row),
            pl.BlockSpec((blk, EMB), row),
            pl.BlockSpec((blk, EMB), row),
            pl.BlockSpec((2 * EMB, HID), rep),
            pl.BlockSpec((1, HID), rep),
            pl.BlockSpec((HID, HID // 2), rep),
            pl.BlockSpec((1, HID // 2), rep),
            pl.BlockSpec((HID // 2, EMB), rep),
            pl.BlockSpec((1, EMB), rep),
            pl.BlockSpec((1, 2 * EMB), rep),
            pl.BlockSpec((1, 1), rep),
        ],
        out_specs=pl.BlockSpec((blk,), lambda i: (i,)),
        out_shape=jax.ShapeDtypeStruct((BATCH,), jnp.float32),
    )(gu, gi, mu, mi, W1, b1, W2, b2, W3, b3, Wo, bo)


def kernel(user, item, gmf_user, gmf_item, mlp_user, mlp_item,
           W1, b1, W2, b2, W3, b3, Wo, bo):
    user = user.astype(jnp.int32)
    item = item.astype(jnp.int32)
    gu, gi, mu, mi = _sc_gather4(user, item, gmf_user, gmf_item,
                                 mlp_user, mlp_item)
    out = _tc_dense(gu, gi, mu, mi,
                    W1, b1.reshape(1, HID),
                    W2, b2.reshape(1, HID // 2),
                    W3, b3.reshape(1, EMB),
                    Wo.reshape(1, 2 * EMB), bo.reshape(1, 1))
    return out
